# baseline (speedup 1.0000x reference)
import functools

import jax
import jax.numpy as jnp
from jax import lax
from jax.experimental import pallas as pl
from jax.experimental.pallas import tpu as pltpu
from jax.experimental.pallas import tpu_sc as plsc

_MIN = -3.0
_MAX = 3.0
_NBINS = 1024
_N = 16777216

_NC, _NS, _L = 2, 16, 16
_NW = _NC * _NS
_PER_W = _N // _NW
_CHUNK = 16384
_NCHUNK = _PER_W // _CHUNK
_NPAIR = _NCHUNK // 2

_INV_RANGE = 1.0 / (_MAX - _MIN)


def _body(x_hbm, theta_hbm, out_hbm, theta_v, xin_v, yout_v,
          sem_in0, sem_in1, sem_out0, sem_out1):
    wid = lax.axis_index("s") * _NC + lax.axis_index("c")
    base = wid * _PER_W
    pltpu.sync_copy(theta_hbm, theta_v)

    def in_copy(chunk, slot, sem):
        return pltpu.make_async_copy(
            x_hbm.at[pl.ds(base + chunk * _CHUNK, _CHUNK)], xin_v.at[slot], sem)

    def out_copy(chunk, slot, sem):
        return pltpu.make_async_copy(
            yout_v.at[slot], out_hbm.at[pl.ds(base + chunk * _CHUNK, _CHUNK)],
            sem)

    def compute(slot):
        @plsc.parallel_loop(0, _CHUNK // _L, unroll=32)
        def _(i):
            xv = xin_v[slot, pl.ds(i * _L, _L)]
            t = (xv - _MIN) * (_INV_RANGE * float(_NBINS))
            t = jnp.minimum(jnp.maximum(t, 0.0), float(_NBINS - 1))
            idx = t.astype(jnp.int32)
            yout_v[slot, pl.ds(i * _L, _L)] = plsc.load_gather(theta_v, [idx])

    in_copy(0, 0, sem_in0).start()
    in_copy(1, 1, sem_in1).start()

    def pair_body(p, carry):
        c0 = 2 * p
        in_copy(c0, 0, sem_in0).wait()
        pl.when(p >= 1)(lambda: out_copy(c0 - 2, 0, sem_out0).wait())
        compute(0)
        out_copy(c0, 0, sem_out0).start()
        pl.when(p < _NPAIR - 1)(lambda: in_copy(c0 + 2, 0, sem_in0).start())
        in_copy(c0 + 1, 1, sem_in1).wait()
        pl.when(p >= 1)(lambda: out_copy(c0 - 1, 1, sem_out1).wait())
        compute(1)
        out_copy(c0 + 1, 1, sem_out1).start()
        pl.when(p < _NPAIR - 1)(lambda: in_copy(c0 + 3, 1, sem_in1).start())
        return carry

    lax.fori_loop(0, _NPAIR, pair_body, 0)
    out_copy(_NCHUNK - 2, 0, sem_out0).wait()
    out_copy(_NCHUNK - 1, 1, sem_out1).wait()


def kernel(x, theta):
    mesh = plsc.VectorSubcoreMesh(core_axis_name="c", subcore_axis_name="s")
    f = functools.partial(
        pl.kernel,
        mesh=mesh,
        out_type=jax.ShapeDtypeStruct((_N,), jnp.float32),
        scratch_types=[
            pltpu.VMEM((_NBINS,), jnp.float32),
            pltpu.VMEM((2, _CHUNK), jnp.float32),
            pltpu.VMEM((2, _CHUNK), jnp.float32),
            pltpu.SemaphoreType.DMA,
            pltpu.SemaphoreType.DMA,
            pltpu.SemaphoreType.DMA,
            pltpu.SemaphoreType.DMA,
        ],
        compiler_params=pltpu.CompilerParams(needs_layout_passes=False),
    )(_body)
    return f(x, theta)


# speedup vs baseline: 1.4562x; 1.4562x over previous
import functools

import jax
import jax.numpy as jnp
from jax import lax
from jax.experimental import pallas as pl
from jax.experimental.pallas import tpu as pltpu
from jax.experimental.pallas import tpu_sc as plsc

_MIN = -3.0
_MAX = 3.0
_NBINS = 1024
_N = 16777216

_NC, _NS, _L = 2, 16, 16
_NW = _NC * _NS
_PER_W = _N // _NW
_CHUNK = 16384
_NCHUNK = _PER_W // _CHUNK
_NPAIR = _NCHUNK // 2

_INV_RANGE = 1.0 / (_MAX - _MIN)


def _body(x_hbm, theta_hbm, out_hbm, theta_v, xin_v, yout_v,
          sem_in0, sem_in1, sem_out0, sem_out1):
    wid = lax.axis_index("s") * _NC + lax.axis_index("c")
    base = wid * _PER_W
    pltpu.sync_copy(theta_hbm, theta_v)

    def in_copy(chunk, slot, sem):
        return pltpu.make_async_copy(
            x_hbm.at[pl.ds(base + chunk * _CHUNK, _CHUNK)], xin_v.at[slot], sem)

    def out_copy(chunk, slot, sem):
        return pltpu.make_async_copy(
            yout_v.at[slot], out_hbm.at[pl.ds(base + chunk * _CHUNK, _CHUNK)],
            sem)

    def compute(slot):
        @plsc.parallel_loop(0, _CHUNK // _L, unroll=16)
        def _(i):
            xv = xin_v[slot, pl.ds(i * _L, _L)]
            t = (xv - _MIN) * (_INV_RANGE * float(_NBINS))
            t = jnp.minimum(jnp.maximum(t, 0.0), float(_NBINS - 1))
            idx = t.astype(jnp.int32)
            yout_v[slot, pl.ds(i * _L, _L)] = plsc.load_gather(theta_v, [idx])

    in_copy(0, 0, sem_in0).start()
    in_copy(1, 1, sem_in1).start()

    def pair_body(p, carry):
        c0 = 2 * p
        in_copy(c0, 0, sem_in0).wait()
        pl.when(p >= 1)(lambda: out_copy(c0 - 2, 0, sem_out0).wait())
        compute(0)
        out_copy(c0, 0, sem_out0).start()
        pl.when(p < _NPAIR - 1)(lambda: in_copy(c0 + 2, 0, sem_in0).start())
        in_copy(c0 + 1, 1, sem_in1).wait()
        pl.when(p >= 1)(lambda: out_copy(c0 - 1, 1, sem_out1).wait())
        compute(1)
        out_copy(c0 + 1, 1, sem_out1).start()
        pl.when(p < _NPAIR - 1)(lambda: in_copy(c0 + 3, 1, sem_in1).start())
        return carry

    lax.fori_loop(0, _NPAIR, pair_body, 0)
    out_copy(_NCHUNK - 2, 0, sem_out0).wait()
    out_copy(_NCHUNK - 1, 1, sem_out1).wait()


def kernel(x, theta):
    mesh = plsc.VectorSubcoreMesh(core_axis_name="c", subcore_axis_name="s")
    f = functools.partial(
        pl.kernel,
        mesh=mesh,
        out_type=jax.ShapeDtypeStruct((_N,), jnp.float32),
        scratch_types=[
            pltpu.VMEM((_NBINS,), jnp.float32),
            pltpu.VMEM((2, _CHUNK), jnp.float32),
            pltpu.VMEM((2, _CHUNK), jnp.float32),
            pltpu.SemaphoreType.DMA,
            pltpu.SemaphoreType.DMA,
            pltpu.SemaphoreType.DMA,
            pltpu.SemaphoreType.DMA,
        ],
        compiler_params=pltpu.CompilerParams(needs_layout_passes=False, use_tc_tiling_on_sc=False),
    )(_body)
    return f(x, theta)


# speedup vs baseline: 1.4647x; 1.0058x over previous
import functools

import jax
import jax.numpy as jnp
from jax import lax
from jax.experimental import pallas as pl
from jax.experimental.pallas import tpu as pltpu
from jax.experimental.pallas import tpu_sc as plsc

_MIN = -3.0
_MAX = 3.0
_NBINS = 1024
_N = 16777216

_NC, _NS, _L = 2, 16, 16
_NW = _NC * _NS
_PER_W = _N // _NW
_CHUNK = 16384
_NCHUNK = _PER_W // _CHUNK
_NPAIR = _NCHUNK // 2

_INV_RANGE = 1.0 / (_MAX - _MIN)


def _body(x_hbm, theta_hbm, out_hbm, theta_v, xin_v, yout_v,
          sem_in0, sem_in1, sem_out0, sem_out1):
    wid = lax.axis_index("s") * _NC + lax.axis_index("c")
    base = wid * _PER_W

    def in_copy(chunk, slot, sem):
        return pltpu.make_async_copy(
            x_hbm.at[pl.ds(base + chunk * _CHUNK, _CHUNK)], xin_v.at[slot], sem)

    def out_copy(chunk, slot, sem):
        return pltpu.make_async_copy(
            yout_v.at[slot], out_hbm.at[pl.ds(base + chunk * _CHUNK, _CHUNK)],
            sem)

    def compute(slot):
        @plsc.parallel_loop(0, _CHUNK // _L, unroll=16)
        def _(i):
            xv = xin_v[slot, pl.ds(i * _L, _L)]
            t = (xv - _MIN) * (_INV_RANGE * float(_NBINS))
            t = jnp.minimum(jnp.maximum(t, 0.0), float(_NBINS - 1))
            idx = t.astype(jnp.int32)
            yout_v[slot, pl.ds(i * _L, _L)] = plsc.load_gather(theta_v, [idx])

    in_copy(0, 0, sem_in0).start()
    in_copy(1, 1, sem_in1).start()
    pltpu.sync_copy(theta_hbm, theta_v)

    def pair_body(p, carry):
        c0 = 2 * p
        in_copy(c0, 0, sem_in0).wait()
        pl.when(p >= 1)(lambda: out_copy(c0 - 2, 0, sem_out0).wait())
        compute(0)
        out_copy(c0, 0, sem_out0).start()
        pl.when(p < _NPAIR - 1)(lambda: in_copy(c0 + 2, 0, sem_in0).start())
        in_copy(c0 + 1, 1, sem_in1).wait()
        pl.when(p >= 1)(lambda: out_copy(c0 - 1, 1, sem_out1).wait())
        compute(1)
        out_copy(c0 + 1, 1, sem_out1).start()
        pl.when(p < _NPAIR - 1)(lambda: in_copy(c0 + 3, 1, sem_in1).start())
        return carry

    lax.fori_loop(0, _NPAIR, pair_body, 0)
    out_copy(_NCHUNK - 2, 0, sem_out0).wait()
    out_copy(_NCHUNK - 1, 1, sem_out1).wait()


def kernel(x, theta):
    mesh = plsc.VectorSubcoreMesh(core_axis_name="c", subcore_axis_name="s")
    f = functools.partial(
        pl.kernel,
        mesh=mesh,
        out_type=jax.ShapeDtypeStruct((_N,), jnp.float32),
        scratch_types=[
            pltpu.VMEM((_NBINS,), jnp.float32),
            pltpu.VMEM((2, _CHUNK), jnp.float32),
            pltpu.VMEM((2, _CHUNK), jnp.float32),
            pltpu.SemaphoreType.DMA,
            pltpu.SemaphoreType.DMA,
            pltpu.SemaphoreType.DMA,
            pltpu.SemaphoreType.DMA,
        ],
        compiler_params=pltpu.CompilerParams(needs_layout_passes=False, use_tc_tiling_on_sc=False, disable_bounds_checks=True, disable_semaphore_checks=True, skip_device_barrier=True),
    )(_body)
    return f(x, theta)


# speedup vs baseline: 1.4696x; 1.0034x over previous
import functools

import jax
import jax.numpy as jnp
from jax import lax
from jax.experimental import pallas as pl
from jax.experimental.pallas import tpu as pltpu
from jax.experimental.pallas import tpu_sc as plsc

_MIN = -3.0
_MAX = 3.0
_NBINS = 1024
_N = 16777216

_NC, _NS, _L = 2, 16, 16
_NW = _NC * _NS
_PER_W = _N // _NW
_CHUNK = 16384
_NCHUNK = _PER_W // _CHUNK
_NPAIR = _NCHUNK // 2

_INV_RANGE = 1.0 / (_MAX - _MIN)


def _body(x_hbm, theta_hbm, out_hbm, theta_v, xin_v, yout_v,
          sem_in0, sem_in1, sem_out0, sem_out1):
    wid = lax.axis_index("s") * _NC + lax.axis_index("c")
    base = wid * _PER_W

    def in_copy(chunk, slot, sem):
        return pltpu.make_async_copy(
            x_hbm.at[pl.ds(base + chunk * _CHUNK, _CHUNK)], xin_v.at[slot], sem)

    def out_copy(chunk, slot, sem):
        return pltpu.make_async_copy(
            yout_v.at[slot], out_hbm.at[pl.ds(base + chunk * _CHUNK, _CHUNK)],
            sem)

    def compute(slot):
        @plsc.parallel_loop(0, _CHUNK // _L, unroll=8)
        def _(i):
            xv = xin_v[slot, pl.ds(i * _L, _L)]
            t = (xv - _MIN) * (_INV_RANGE * float(_NBINS))
            t = jnp.minimum(jnp.maximum(t, 0.0), float(_NBINS - 1))
            idx = t.astype(jnp.int32)
            yout_v[slot, pl.ds(i * _L, _L)] = plsc.load_gather(theta_v, [idx])

    in_copy(0, 0, sem_in0).start()
    in_copy(1, 1, sem_in1).start()
    pltpu.sync_copy(theta_hbm, theta_v)

    def pair_body(p, carry):
        c0 = 2 * p
        in_copy(c0, 0, sem_in0).wait()
        pl.when(p >= 1)(lambda: out_copy(c0 - 2, 0, sem_out0).wait())
        compute(0)
        out_copy(c0, 0, sem_out0).start()
        pl.when(p < _NPAIR - 1)(lambda: in_copy(c0 + 2, 0, sem_in0).start())
        in_copy(c0 + 1, 1, sem_in1).wait()
        pl.when(p >= 1)(lambda: out_copy(c0 - 1, 1, sem_out1).wait())
        compute(1)
        out_copy(c0 + 1, 1, sem_out1).start()
        pl.when(p < _NPAIR - 1)(lambda: in_copy(c0 + 3, 1, sem_in1).start())
        return carry

    lax.fori_loop(0, _NPAIR, pair_body, 0)
    out_copy(_NCHUNK - 2, 0, sem_out0).wait()
    out_copy(_NCHUNK - 1, 1, sem_out1).wait()


def kernel(x, theta):
    mesh = plsc.VectorSubcoreMesh(core_axis_name="c", subcore_axis_name="s")
    f = functools.partial(
        pl.kernel,
        mesh=mesh,
        out_type=jax.ShapeDtypeStruct((_N,), jnp.float32),
        scratch_types=[
            pltpu.VMEM((_NBINS,), jnp.float32),
            pltpu.VMEM((2, _CHUNK), jnp.float32),
            pltpu.VMEM((2, _CHUNK), jnp.float32),
            pltpu.SemaphoreType.DMA,
            pltpu.SemaphoreType.DMA,
            pltpu.SemaphoreType.DMA,
            pltpu.SemaphoreType.DMA,
        ],
        compiler_params=pltpu.CompilerParams(needs_layout_passes=False, use_tc_tiling_on_sc=False, disable_bounds_checks=True, disable_semaphore_checks=True, skip_device_barrier=True),
    )(_body)
    return f(x, theta)
